# baseline (speedup 1.0000x reference)
import jax
import jax.numpy as jnp
from jax import lax
from jax.experimental import pallas as pl
from jax.experimental.pallas import tpu as pltpu
from jax.experimental.pallas import tpu_sc as plsc

N = 10000
E = 320000
D_IN = 128
D_HID = 256
D_OUT = 128

NC = 2
NS = 16
CH = 128
NCHUNK = 2560
E_PAD = NCHUNK * CH
CPS = NCHUNK // NS
CPT = NCHUNK // (NC * NS)
IB = 16
KOP = 2
ROP = KOP * CH
OPS_COL = NCHUNK // KOP // NS
OPS_EDGE = NCHUNK // KOP // (NC * NS)
NPAD = 10240
RPS = NPAD // NS
BM = 1000
DH = 128


def _mesh():
    return plsc.VectorSubcoreMesh(
        core_axis_name="c", subcore_axis_name="s", num_cores=NC,
        num_subcores=NS)


def _deg_body(dst_h, ones_h, zeros_h, out_h, didx, ones_l, deg_sp):
    c = lax.axis_index("c")
    s = lax.axis_index("s")
    w = c * NS + s
    pltpu.sync_copy(dst_h.at[pl.ds(w * CPT, CPT)], didx)
    pltpu.sync_copy(ones_h, ones_l)
    pltpu.sync_copy(zeros_h.at[pl.ds(s * RPS, RPS)],
                    deg_sp.at[pl.ds(s * RPS, RPS)])
    plsc.subcore_barrier()

    def body(j, carry):
        pltpu.sync_copy(ones_l, deg_sp.at[didx.at[j]], add=True)
        return carry

    lax.fori_loop(0, CPT, body, 0)
    plsc.subcore_barrier()
    pltpu.sync_copy(deg_sp.at[pl.ds(s * RPS, RPS)],
                    out_h.at[c, pl.ds(s * RPS, RPS)])


_deg_call = pl.kernel(
    _deg_body,
    out_type=jax.ShapeDtypeStruct((NC, NPAD), jnp.float32),
    mesh=_mesh(),
    scratch_types=[
        pltpu.VMEM((CPT, CH), jnp.int32),
        pltpu.VMEM((CH,), jnp.float32),
        pltpu.VMEM_SHARED((NPAD,), jnp.float32),
    ],
)


def _agg_pipeline(g_h, sidx, didx, buf, acc, nops, op_base_fn,
                  src_flat_view, dst_flat):
    def step(j, carry):
        o = op_base_fn(j)
        pltpu.sync_copy(src_flat_view.at[pl.ds(o * ROP, ROP)], sidx)
        pltpu.sync_copy(dst_flat.at[pl.ds(o * ROP, ROP)], didx)
        pltpu.sync_copy(g_h.at[sidx], buf)
        pltpu.sync_copy(buf, acc.at[didx], add=True)
        return carry

    lax.fori_loop(0, nops, step, 0)


def _agg_col_body(g_h, srcx_h, dst_h, zeros_h, out_h,
                  sidx, didx, buf, acc):
    c = lax.axis_index("c")
    s = lax.axis_index("s")
    pltpu.sync_copy(zeros_h.at[pl.ds(s * RPS, RPS)],
                    acc.at[pl.ds(s * RPS, RPS)])
    plsc.subcore_barrier()
    _agg_pipeline(g_h, sidx, didx, buf, acc,
                  OPS_COL, lambda j: s * OPS_COL + j,
                  srcx_h.at[c], dst_h)
    plsc.subcore_barrier()
    pltpu.sync_copy(acc.at[pl.ds(s * RPS, RPS)],
                    out_h.at[c, pl.ds(s * RPS, RPS)])


def _agg_edge_body(g_h, src_h, dst_h, zeros_h, out_h,
                   sidx, didx, buf, acc):
    c = lax.axis_index("c")
    s = lax.axis_index("s")
    w = c * NS + s
    pltpu.sync_copy(zeros_h.at[pl.ds(s * RPS, RPS)],
                    acc.at[pl.ds(s * RPS, RPS)])
    plsc.subcore_barrier()
    _agg_pipeline(g_h, sidx, didx, buf, acc,
                  OPS_EDGE, lambda j: w * OPS_EDGE + j,
                  src_h, dst_h)
    plsc.subcore_barrier()
    pltpu.sync_copy(acc.at[pl.ds(s * RPS, RPS)],
                    out_h.at[c, pl.ds(s * RPS, RPS)])


def _make_agg(body, table_rows):
    return pl.kernel(
        body,
        out_type=jax.ShapeDtypeStruct((NC, NPAD, DH), jnp.float32),
        mesh=_mesh(),
        scratch_types=[
            pltpu.VMEM((ROP,), jnp.int32),
            pltpu.VMEM((ROP,), jnp.int32),
            pltpu.VMEM((ROP, DH), jnp.float32),
            pltpu.VMEM_SHARED((NPAD, DH), jnp.float32),
        ],
    )


_agg_col = _make_agg(_agg_col_body, 2 * N)
_agg_edge = _make_agg(_agg_edge_body, N)


def _tc1_body(deg_ref, x_ref, w_ref, h_ref, g_ref):
    deg = deg_ref[...]
    dis = lax.rsqrt(deg[:, 0] + deg[:, 1] + 1.0)
    h = jnp.dot(x_ref[...], w_ref[...], preferred_element_type=jnp.float32,
                precision=lax.Precision.HIGHEST)
    g = h * dis[:, None]
    h_ref[...] = h
    g_ref[0, :, :] = g[:, :D_HID // 2]
    g_ref[1, :, :] = g[:, D_HID // 2:]


_tc1_call = pl.pallas_call(
    _tc1_body,
    grid=(N // BM,),
    in_specs=[
        pl.BlockSpec((BM, 2), lambda i: (i, 0)),
        pl.BlockSpec((BM, D_IN), lambda i: (i, 0)),
        pl.BlockSpec((D_IN, D_HID), lambda i: (0, 0)),
    ],
    out_specs=[
        pl.BlockSpec((BM, D_HID), lambda i: (i, 0)),
        pl.BlockSpec((NC, BM, D_HID // 2), lambda i: (0, i, 0)),
    ],
    out_shape=[
        jax.ShapeDtypeStruct((N, D_HID), jnp.float32),
        jax.ShapeDtypeStruct((NC, N, D_HID // 2), jnp.float32),
    ],
)


def _tc2_body(deg_ref, acc_ref, h1_ref, b1_ref, w2_ref, h2_ref, g2_ref):
    deg = deg_ref[...]
    dis = lax.rsqrt(deg[:, 0] + deg[:, 1] + 1.0)
    accf = jnp.concatenate([acc_ref[0, :, :], acc_ref[1, :, :]], axis=-1)
    z = accf * dis[:, None] + h1_ref[...] * (dis * dis)[:, None] + b1_ref[...]
    z = jnp.maximum(z, 0.0)
    h2 = jnp.dot(z, w2_ref[...], preferred_element_type=jnp.float32,
                 precision=lax.Precision.HIGHEST)
    h2_ref[...] = h2
    g2_ref[...] = h2 * dis[:, None]


_tc2_call = pl.pallas_call(
    _tc2_body,
    grid=(N // BM,),
    in_specs=[
        pl.BlockSpec((BM, 2), lambda i: (i, 0)),
        pl.BlockSpec((NC, BM, D_HID // 2), lambda i: (0, i, 0)),
        pl.BlockSpec((BM, D_HID), lambda i: (i, 0)),
        pl.BlockSpec((1, D_HID), lambda i: (0, 0)),
        pl.BlockSpec((D_HID, D_OUT), lambda i: (0, 0)),
    ],
    out_specs=[
        pl.BlockSpec((BM, D_OUT), lambda i: (i, 0)),
        pl.BlockSpec((BM, D_OUT), lambda i: (i, 0)),
    ],
    out_shape=[
        jax.ShapeDtypeStruct((N, D_OUT), jnp.float32),
        jax.ShapeDtypeStruct((N, D_OUT), jnp.float32),
    ],
)


def _tc3_body(deg_ref, acc_ref, h2_ref, b2_ref, o_ref):
    deg = deg_ref[...]
    dis = lax.rsqrt(deg[:, 0] + deg[:, 1] + 1.0)
    accf = acc_ref[0, :, :] + acc_ref[1, :, :]
    o_ref[...] = (accf * dis[:, None]
                  + h2_ref[...] * (dis * dis)[:, None] + b2_ref[...])


_tc3_call = pl.pallas_call(
    _tc3_body,
    grid=(N // BM,),
    in_specs=[
        pl.BlockSpec((BM, 2), lambda i: (i, 0)),
        pl.BlockSpec((NC, BM, D_OUT), lambda i: (0, i, 0)),
        pl.BlockSpec((BM, D_OUT), lambda i: (i, 0)),
        pl.BlockSpec((1, D_OUT), lambda i: (0, 0)),
    ],
    out_specs=pl.BlockSpec((BM, D_OUT), lambda i: (i, 0)),
    out_shape=jax.ShapeDtypeStruct((N, D_OUT), jnp.float32),
)


@jax.jit
def kernel(x, edge_index, W1, b1, W2, b2):
    src = edge_index[0].astype(jnp.int32)
    dst = edge_index[1].astype(jnp.int32)
    pad = E_PAD - E
    dst_p = jnp.concatenate(
        [dst, jnp.full((pad,), N, jnp.int32)]).reshape(NCHUNK, CH)
    dst_f = dst_p.reshape(E_PAD)
    src_f = jnp.concatenate([src, jnp.zeros((pad,), jnp.int32)])
    srcx = jnp.stack([src_f, src_f + N])

    ones_h = jnp.ones((CH,), jnp.float32)
    zeros_deg = jnp.zeros((NPAD,), jnp.float32)
    zeros128 = jnp.zeros((NPAD, DH), jnp.float32)

    degp = _deg_call(dst_p, ones_h, zeros_deg)
    deg_nt = jnp.transpose(degp[:, :N], (1, 0))

    h1, g1 = _tc1_call(deg_nt, x, W1)
    acc1 = _agg_col(g1.reshape(NC * N, DH), srcx, dst_f, zeros128)[:, :N]
    h2, g2 = _tc2_call(deg_nt, acc1, h1, b1.reshape(1, D_HID), W2)
    acc2 = _agg_edge(g2, src_f, dst_f, zeros128)[:, :N]
    out = _tc3_call(deg_nt, acc2, h2, b2.reshape(1, D_OUT))
    return out


# speedup vs baseline: 1.0635x; 1.0635x over previous
import jax
import jax.numpy as jnp
from jax import lax
from jax.experimental import pallas as pl
from jax.experimental.pallas import tpu as pltpu
from jax.experimental.pallas import tpu_sc as plsc

N = 10000
E = 320000
D_IN = 128
D_HID = 256
D_OUT = 128

NC = 2
NS = 16
CH = 128
NCHUNK = 2560
E_PAD = NCHUNK * CH
CPS = NCHUNK // NS
CPT = NCHUNK // (NC * NS)
IB = 16
NPAD = 10240
RPS = NPAD // NS
BM = 1000
DH = 128


def _mesh():
    return plsc.VectorSubcoreMesh(
        core_axis_name="c", subcore_axis_name="s", num_cores=NC,
        num_subcores=NS)


def _deg_body(dst_h, ones_h, zeros_h, out_h, didx, ones_l, deg_sp):
    c = lax.axis_index("c")
    s = lax.axis_index("s")
    w = c * NS + s
    pltpu.sync_copy(dst_h.at[pl.ds(w * CPT, CPT)], didx)
    pltpu.sync_copy(ones_h, ones_l)
    pltpu.sync_copy(zeros_h.at[pl.ds(s * RPS, RPS)],
                    deg_sp.at[pl.ds(s * RPS, RPS)])
    plsc.subcore_barrier()

    def body(j, carry):
        pltpu.sync_copy(ones_l, deg_sp.at[didx.at[j]], add=True)
        return carry

    lax.fori_loop(0, CPT, body, 0)
    plsc.subcore_barrier()
    pltpu.sync_copy(deg_sp.at[pl.ds(s * RPS, RPS)],
                    out_h.at[c, pl.ds(s * RPS, RPS)])


_deg_call = pl.kernel(
    _deg_body,
    out_type=jax.ShapeDtypeStruct((NC, NPAD), jnp.float32),
    mesh=_mesh(),
    scratch_types=[
        pltpu.VMEM((CPT, CH), jnp.int32),
        pltpu.VMEM((CH,), jnp.float32),
        pltpu.VMEM_SHARED((NPAD,), jnp.float32),
    ],
)


def _agg_pipeline(g_h, sidx, didx, buf, sem_g, sem_s, acc, nblk, blk_base_fn,
                  srcx_view, dst_h):
    def block(k, carry):
        base = blk_base_fn(k)
        pltpu.sync_copy(srcx_view.at[pl.ds(base, IB)], sidx)
        pltpu.sync_copy(dst_h.at[pl.ds(base, IB)], didx)
        pltpu.make_async_copy(g_h.at[sidx.at[0]], buf.at[0], sem_g).start()

        def step(j2, carry2):
            for b in range(2):
                j = j2 * 2 + b
                pltpu.make_async_copy(
                    g_h.at[sidx.at[j]], buf.at[b], sem_g).wait()
                pltpu.make_async_copy(
                    buf.at[b], acc.at[didx.at[j]], sem_s).start(add=True)

                @pl.when(j + 1 < IB)
                def _():
                    @pl.when(j >= 1)
                    def _():
                        pltpu.make_async_copy(
                            buf.at[1 - b], acc.at[didx.at[j - 1]],
                            sem_s).wait()

                    pltpu.make_async_copy(
                        g_h.at[sidx.at[j + 1]], buf.at[1 - b], sem_g).start()
            return carry2

        lax.fori_loop(0, IB // 2, step, 0)
        pltpu.make_async_copy(
            buf.at[1], acc.at[didx.at[IB - 1]], sem_s).wait()
        return carry

    lax.fori_loop(0, nblk, block, 0)


def _agg_col_body(g_h, srcx_h, dst_h, zeros_h, out_h,
                  sidx, didx, buf, sem_g, sem_s, acc):
    c = lax.axis_index("c")
    s = lax.axis_index("s")
    pltpu.sync_copy(zeros_h.at[pl.ds(s * RPS, RPS)],
                    acc.at[pl.ds(s * RPS, RPS)])
    plsc.subcore_barrier()
    _agg_pipeline(g_h, sidx, didx, buf, sem_g, sem_s, acc,
                  CPS // IB, lambda k: s * CPS + k * IB,
                  srcx_h.at[c], dst_h)
    plsc.subcore_barrier()
    pltpu.sync_copy(acc.at[pl.ds(s * RPS, RPS)],
                    out_h.at[c, pl.ds(s * RPS, RPS)])


def _agg_edge_body(g_h, src_h, dst_h, zeros_h, out_h,
                   sidx, didx, buf, sem_g, sem_s, acc):
    c = lax.axis_index("c")
    s = lax.axis_index("s")
    w = c * NS + s
    pltpu.sync_copy(zeros_h.at[pl.ds(s * RPS, RPS)],
                    acc.at[pl.ds(s * RPS, RPS)])
    plsc.subcore_barrier()
    _agg_pipeline(g_h, sidx, didx, buf, sem_g, sem_s, acc,
                  CPT // IB, lambda k: w * CPT + k * IB,
                  src_h, dst_h)
    plsc.subcore_barrier()
    pltpu.sync_copy(acc.at[pl.ds(s * RPS, RPS)],
                    out_h.at[c, pl.ds(s * RPS, RPS)])


def _make_agg(body, table_rows):
    return pl.kernel(
        body,
        out_type=jax.ShapeDtypeStruct((NC, NPAD, DH), jnp.float32),
        mesh=_mesh(),
        scratch_types=[
            pltpu.VMEM((IB, CH), jnp.int32),
            pltpu.VMEM((IB, CH), jnp.int32),
            pltpu.VMEM((2, CH, DH), jnp.float32),
            pltpu.SemaphoreType.DMA,
            pltpu.SemaphoreType.DMA,
            pltpu.VMEM_SHARED((NPAD, DH), jnp.float32),
        ],
    )


_agg_col = _make_agg(_agg_col_body, 2 * N)
_agg_edge = _make_agg(_agg_edge_body, N)


def _tc1_body(deg_ref, x_ref, w_ref, h_ref, g_ref):
    deg = deg_ref[...]
    dis = lax.rsqrt(deg[:, 0] + deg[:, 1] + 1.0)
    h = jnp.dot(x_ref[...], w_ref[...], preferred_element_type=jnp.float32,
                precision=lax.Precision.HIGHEST)
    g = h * dis[:, None]
    h_ref[...] = h
    g_ref[0, :, :] = g[:, :D_HID // 2]
    g_ref[1, :, :] = g[:, D_HID // 2:]


_tc1_call = pl.pallas_call(
    _tc1_body,
    grid=(N // BM,),
    in_specs=[
        pl.BlockSpec((BM, 2), lambda i: (i, 0)),
        pl.BlockSpec((BM, D_IN), lambda i: (i, 0)),
        pl.BlockSpec((D_IN, D_HID), lambda i: (0, 0)),
    ],
    out_specs=[
        pl.BlockSpec((BM, D_HID), lambda i: (i, 0)),
        pl.BlockSpec((NC, BM, D_HID // 2), lambda i: (0, i, 0)),
    ],
    out_shape=[
        jax.ShapeDtypeStruct((N, D_HID), jnp.float32),
        jax.ShapeDtypeStruct((NC, N, D_HID // 2), jnp.float32),
    ],
)


def _tc2_body(deg_ref, acc_ref, h1_ref, b1_ref, w2_ref, h2_ref, g2_ref):
    deg = deg_ref[...]
    dis = lax.rsqrt(deg[:, 0] + deg[:, 1] + 1.0)
    accf = jnp.concatenate([acc_ref[0, :, :], acc_ref[1, :, :]], axis=-1)
    z = accf * dis[:, None] + h1_ref[...] * (dis * dis)[:, None] + b1_ref[...]
    z = jnp.maximum(z, 0.0)
    h2 = jnp.dot(z, w2_ref[...], preferred_element_type=jnp.float32,
                 precision=lax.Precision.HIGHEST)
    h2_ref[...] = h2
    g2_ref[...] = h2 * dis[:, None]


_tc2_call = pl.pallas_call(
    _tc2_body,
    grid=(N // BM,),
    in_specs=[
        pl.BlockSpec((BM, 2), lambda i: (i, 0)),
        pl.BlockSpec((NC, BM, D_HID // 2), lambda i: (0, i, 0)),
        pl.BlockSpec((BM, D_HID), lambda i: (i, 0)),
        pl.BlockSpec((1, D_HID), lambda i: (0, 0)),
        pl.BlockSpec((D_HID, D_OUT), lambda i: (0, 0)),
    ],
    out_specs=[
        pl.BlockSpec((BM, D_OUT), lambda i: (i, 0)),
        pl.BlockSpec((BM, D_OUT), lambda i: (i, 0)),
    ],
    out_shape=[
        jax.ShapeDtypeStruct((N, D_OUT), jnp.float32),
        jax.ShapeDtypeStruct((N, D_OUT), jnp.float32),
    ],
)


def _tc3_body(deg_ref, acc_ref, h2_ref, b2_ref, o_ref):
    deg = deg_ref[...]
    dis = lax.rsqrt(deg[:, 0] + deg[:, 1] + 1.0)
    accf = acc_ref[0, :, :] + acc_ref[1, :, :]
    o_ref[...] = (accf * dis[:, None]
                  + h2_ref[...] * (dis * dis)[:, None] + b2_ref[...])


_tc3_call = pl.pallas_call(
    _tc3_body,
    grid=(N // BM,),
    in_specs=[
        pl.BlockSpec((BM, 2), lambda i: (i, 0)),
        pl.BlockSpec((NC, BM, D_OUT), lambda i: (0, i, 0)),
        pl.BlockSpec((BM, D_OUT), lambda i: (i, 0)),
        pl.BlockSpec((1, D_OUT), lambda i: (0, 0)),
    ],
    out_specs=pl.BlockSpec((BM, D_OUT), lambda i: (i, 0)),
    out_shape=jax.ShapeDtypeStruct((N, D_OUT), jnp.float32),
)


@jax.jit
def kernel(x, edge_index, W1, b1, W2, b2):
    src = edge_index[0].astype(jnp.int32)
    dst = edge_index[1].astype(jnp.int32)
    pad = E_PAD - E
    dst_p = jnp.concatenate(
        [dst, jnp.full((pad,), N, jnp.int32)]).reshape(NCHUNK, CH)
    src_p = jnp.concatenate(
        [src, jnp.zeros((pad,), jnp.int32)]).reshape(NCHUNK, CH)
    srcx = jnp.stack([src_p, src_p + N])

    ones_h = jnp.ones((CH,), jnp.float32)
    zeros_deg = jnp.zeros((NPAD,), jnp.float32)
    zeros128 = jnp.zeros((NPAD, DH), jnp.float32)

    degp = _deg_call(dst_p, ones_h, zeros_deg)
    deg_nt = jnp.transpose(degp[:, :N], (1, 0))

    h1, g1 = _tc1_call(deg_nt, x, W1)
    acc1 = _agg_col(g1.reshape(NC * N, DH), srcx, dst_p, zeros128)
    h2, g2 = _tc2_call(deg_nt, acc1, h1, b1.reshape(1, D_HID), W2)
    acc2 = _agg_edge(g2, src_p, dst_p, zeros128)
    out = _tc3_call(deg_nt, acc2, h2, b2.reshape(1, D_OUT))
    return out


# speedup vs baseline: 1.2337x; 1.1601x over previous
import jax
import jax.numpy as jnp
from jax import lax
from jax.experimental import pallas as pl
from jax.experimental.pallas import tpu as pltpu
from jax.experimental.pallas import tpu_sc as plsc

N = 10000
E = 320000
D_IN = 128
D_HID = 256
D_OUT = 128

NC = 2
NS = 16
CH = 128
NCHUNK = 2560
E_PAD = NCHUNK * CH
CPT = NCHUNK // (NC * NS)
IB = 16
NPAD = 10240
RPS = NPAD // NS
BM = 1000
DH = 128


def _mesh():
    return plsc.VectorSubcoreMesh(
        core_axis_name="c", subcore_axis_name="s", num_cores=NC,
        num_subcores=NS)


def _deg_body(dst_h, ones_h, zeros_h, out_h, didx, ones_l, deg_sp):
    c = lax.axis_index("c")
    s = lax.axis_index("s")
    w = c * NS + s
    pltpu.sync_copy(dst_h.at[pl.ds(w * CPT, CPT)], didx)
    pltpu.sync_copy(ones_h, ones_l)
    pltpu.sync_copy(zeros_h.at[pl.ds(s * RPS, RPS)],
                    deg_sp.at[pl.ds(s * RPS, RPS)])
    plsc.subcore_barrier()

    def body(j, carry):
        pltpu.sync_copy(ones_l, deg_sp.at[didx.at[j]], add=True)
        return carry

    lax.fori_loop(0, CPT, body, 0)
    plsc.subcore_barrier()
    pltpu.sync_copy(deg_sp.at[pl.ds(s * RPS, RPS)],
                    out_h.at[c, pl.ds(s * RPS, RPS)])


_deg_call = pl.kernel(
    _deg_body,
    out_type=jax.ShapeDtypeStruct((NC, NPAD), jnp.float32),
    mesh=_mesh(),
    scratch_types=[
        pltpu.VMEM((CPT, CH), jnp.int32),
        pltpu.VMEM((CH,), jnp.float32),
        pltpu.VMEM_SHARED((NPAD,), jnp.float32),
    ],
)


def _agg_edge_body(g_h, src_h, dst_h, zeros_h, out_h,
                   sidx, didx, buf, sem_g, sem_s, acc):
    c = lax.axis_index("c")
    s = lax.axis_index("s")
    w = c * NS + s
    pltpu.sync_copy(zeros_h.at[pl.ds(s * RPS, RPS)],
                    acc.at[pl.ds(s * RPS, RPS)])
    plsc.subcore_barrier()

    def block(k, carry):
        base = w * CPT + k * IB
        pltpu.sync_copy(src_h.at[pl.ds(base, IB)], sidx)
        pltpu.sync_copy(dst_h.at[pl.ds(base, IB)], didx)
        pltpu.make_async_copy(g_h.at[sidx.at[0]], buf.at[0], sem_g).start()

        def step(j2, carry2):
            for b in range(2):
                j = j2 * 2 + b
                pltpu.make_async_copy(
                    g_h.at[sidx.at[j]], buf.at[b], sem_g).wait()
                pltpu.make_async_copy(
                    buf.at[b], acc.at[didx.at[j]], sem_s).start(add=True)

                @pl.when(j + 1 < IB)
                def _():
                    @pl.when(j >= 1)
                    def _():
                        pltpu.make_async_copy(
                            buf.at[1 - b], acc.at[didx.at[j - 1]],
                            sem_s).wait()

                    pltpu.make_async_copy(
                        g_h.at[sidx.at[j + 1]], buf.at[1 - b], sem_g).start()
            return carry2

        lax.fori_loop(0, IB // 2, step, 0)
        pltpu.make_async_copy(
            buf.at[1], acc.at[didx.at[IB - 1]], sem_s).wait()
        return carry

    lax.fori_loop(0, CPT // IB, block, 0)
    plsc.subcore_barrier()
    pltpu.sync_copy(acc.at[pl.ds(s * RPS, RPS)],
                    out_h.at[c, pl.ds(s * RPS, RPS)])


_agg_edge = pl.kernel(
    _agg_edge_body,
    out_type=jax.ShapeDtypeStruct((NC, NPAD, DH), jnp.float32),
    mesh=_mesh(),
    scratch_types=[
        pltpu.VMEM((IB, CH), jnp.int32),
        pltpu.VMEM((IB, CH), jnp.int32),
        pltpu.VMEM((2, CH, DH), jnp.float32),
        pltpu.SemaphoreType.DMA,
        pltpu.SemaphoreType.DMA,
        pltpu.VMEM_SHARED((NPAD, DH), jnp.float32),
    ],
)


def _tc1_body(deg_ref, x_ref, w_ref, h_ref, gx_ref):
    deg = deg_ref[...]
    dis = lax.rsqrt(deg[:, 0] + deg[:, 1] + 1.0)
    x = x_ref[...]
    h_ref[...] = jnp.dot(x, w_ref[...], preferred_element_type=jnp.float32,
                         precision=lax.Precision.HIGHEST)
    gx_ref[...] = x * dis[:, None]


_tc1_call = pl.pallas_call(
    _tc1_body,
    grid=(N // BM,),
    in_specs=[
        pl.BlockSpec((BM, 2), lambda i: (i, 0)),
        pl.BlockSpec((BM, D_IN), lambda i: (i, 0)),
        pl.BlockSpec((D_IN, D_HID), lambda i: (0, 0)),
    ],
    out_specs=[
        pl.BlockSpec((BM, D_HID), lambda i: (i, 0)),
        pl.BlockSpec((BM, D_IN), lambda i: (i, 0)),
    ],
    out_shape=[
        jax.ShapeDtypeStruct((N, D_HID), jnp.float32),
        jax.ShapeDtypeStruct((N, D_IN), jnp.float32),
    ],
)


def _tc2_body(deg_ref, acc_ref, h1_ref, b1_ref, w1_ref, w2_ref,
              h2_ref, g2_ref):
    deg = deg_ref[...]
    dis = lax.rsqrt(deg[:, 0] + deg[:, 1] + 1.0)
    accx = acc_ref[0, :, :] + acc_ref[1, :, :]
    hagg = jnp.dot(accx, w1_ref[...], preferred_element_type=jnp.float32,
                   precision=lax.Precision.HIGHEST)
    z = hagg * dis[:, None] + h1_ref[...] * (dis * dis)[:, None] + b1_ref[...]
    z = jnp.maximum(z, 0.0)
    h2 = jnp.dot(z, w2_ref[...], preferred_element_type=jnp.float32,
                 precision=lax.Precision.HIGHEST)
    h2_ref[...] = h2
    g2_ref[...] = h2 * dis[:, None]


_tc2_call = pl.pallas_call(
    _tc2_body,
    grid=(N // BM,),
    in_specs=[
        pl.BlockSpec((BM, 2), lambda i: (i, 0)),
        pl.BlockSpec((NC, BM, D_IN), lambda i: (0, i, 0)),
        pl.BlockSpec((BM, D_HID), lambda i: (i, 0)),
        pl.BlockSpec((1, D_HID), lambda i: (0, 0)),
        pl.BlockSpec((D_IN, D_HID), lambda i: (0, 0)),
        pl.BlockSpec((D_HID, D_OUT), lambda i: (0, 0)),
    ],
    out_specs=[
        pl.BlockSpec((BM, D_OUT), lambda i: (i, 0)),
        pl.BlockSpec((BM, D_OUT), lambda i: (i, 0)),
    ],
    out_shape=[
        jax.ShapeDtypeStruct((N, D_OUT), jnp.float32),
        jax.ShapeDtypeStruct((N, D_OUT), jnp.float32),
    ],
)


def _tc3_body(deg_ref, acc_ref, h2_ref, b2_ref, o_ref):
    deg = deg_ref[...]
    dis = lax.rsqrt(deg[:, 0] + deg[:, 1] + 1.0)
    accf = acc_ref[0, :, :] + acc_ref[1, :, :]
    o_ref[...] = (accf * dis[:, None]
                  + h2_ref[...] * (dis * dis)[:, None] + b2_ref[...])


_tc3_call = pl.pallas_call(
    _tc3_body,
    grid=(N // BM,),
    in_specs=[
        pl.BlockSpec((BM, 2), lambda i: (i, 0)),
        pl.BlockSpec((NC, BM, D_OUT), lambda i: (0, i, 0)),
        pl.BlockSpec((BM, D_OUT), lambda i: (i, 0)),
        pl.BlockSpec((1, D_OUT), lambda i: (0, 0)),
    ],
    out_specs=pl.BlockSpec((BM, D_OUT), lambda i: (i, 0)),
    out_shape=jax.ShapeDtypeStruct((N, D_OUT), jnp.float32),
)


@jax.jit
def kernel(x, edge_index, W1, b1, W2, b2):
    src = edge_index[0].astype(jnp.int32)
    dst = edge_index[1].astype(jnp.int32)
    pad = E_PAD - E
    dst_p = jnp.concatenate(
        [dst, jnp.full((pad,), N, jnp.int32)]).reshape(NCHUNK, CH)
    src_p = jnp.concatenate(
        [src, jnp.zeros((pad,), jnp.int32)]).reshape(NCHUNK, CH)

    ones_h = jnp.ones((CH,), jnp.float32)
    zeros_deg = jnp.zeros((NPAD,), jnp.float32)
    zeros128 = jnp.zeros((NPAD, DH), jnp.float32)

    degp = _deg_call(dst_p, ones_h, zeros_deg)
    deg_nt = jnp.transpose(degp[:, :N], (1, 0))

    h1, gx = _tc1_call(deg_nt, x, W1)
    acc1 = _agg_edge(gx, src_p, dst_p, zeros128)
    h2, g2 = _tc2_call(deg_nt, acc1, h1, b1.reshape(1, D_HID), W1, W2)
    acc2 = _agg_edge(g2, src_p, dst_p, zeros128)
    out = _tc3_call(deg_nt, acc2, h2, b2.reshape(1, D_OUT))
    return out


# speedup vs baseline: 1.2739x; 1.0326x over previous
import jax
import jax.numpy as jnp
from jax import lax
from jax.experimental import pallas as pl
from jax.experimental.pallas import tpu as pltpu
from jax.experimental.pallas import tpu_sc as plsc

N = 10000
E = 320000
D_IN = 128
D_HID = 256
D_OUT = 128

NC = 2
NS = 16
CH = 128
NCHUNK = 2560
E_PAD = NCHUNK * CH
CPT = NCHUNK // (NC * NS)
IB = 16
NPAD = 10240
RPS = NPAD // NS
BM = 1000
DH = 128


def _mesh():
    return plsc.VectorSubcoreMesh(
        core_axis_name="c", subcore_axis_name="s", num_cores=NC,
        num_subcores=NS)


def _deg_body(dst_h, ones_h, zeros_h, out_h, didx, ones_l, deg_sp):
    c = lax.axis_index("c")
    s = lax.axis_index("s")
    w = c * NS + s
    pltpu.sync_copy(dst_h.at[pl.ds(w * CPT, CPT)], didx)
    pltpu.sync_copy(ones_h, ones_l)
    pltpu.sync_copy(zeros_h.at[pl.ds(s * RPS, RPS)],
                    deg_sp.at[pl.ds(s * RPS, RPS)])
    plsc.subcore_barrier()

    def body(j, carry):
        pltpu.sync_copy(ones_l, deg_sp.at[didx.at[j]], add=True)
        return carry

    lax.fori_loop(0, CPT, body, 0)
    plsc.subcore_barrier()
    pltpu.sync_copy(deg_sp.at[pl.ds(s * RPS, RPS)],
                    out_h.at[c, pl.ds(s * RPS, RPS)])


_deg_call = pl.kernel(
    _deg_body,
    out_type=jax.ShapeDtypeStruct((NC, NPAD), jnp.float32),
    mesh=_mesh(),
    scratch_types=[
        pltpu.VMEM((CPT, CH), jnp.int32),
        pltpu.VMEM((CH,), jnp.float32),
        pltpu.VMEM_SHARED((NPAD,), jnp.float32),
    ],
)


def _agg_edge_body(g_h, src_h, dst_h, zeros_h, out_h,
                   sidx, didx, buf, sem_g, sem_s, acc):
    c = lax.axis_index("c")
    s = lax.axis_index("s")
    w = c * NS + s
    pltpu.sync_copy(zeros_h.at[pl.ds(s * RPS, RPS)],
                    acc.at[pl.ds(s * RPS, RPS)])
    plsc.subcore_barrier()

    def block(k, carry):
        base = w * CPT + k * IB
        pltpu.sync_copy(src_h.at[pl.ds(base, IB)], sidx)
        pltpu.sync_copy(dst_h.at[pl.ds(base, IB)], didx)
        pltpu.make_async_copy(g_h.at[sidx.at[0]], buf.at[0], sem_g).start()
        pltpu.make_async_copy(g_h.at[sidx.at[1]], buf.at[1], sem_g).start()

        def step(j2, carry2):
            for b in range(2):
                j = j2 * 2 + b
                pltpu.make_async_copy(
                    g_h.at[sidx.at[j]], buf.at[b], sem_g).wait()
                pltpu.make_async_copy(
                    buf.at[b], acc.at[didx.at[j]], sem_s).start(add=True)
                pltpu.make_async_copy(
                    buf.at[b], acc.at[didx.at[j]], sem_s).wait()

                @pl.when(j + 2 < IB)
                def _():
                    pltpu.make_async_copy(
                        g_h.at[sidx.at[j + 2]], buf.at[b], sem_g).start()
            return carry2

        lax.fori_loop(0, IB // 2, step, 0)
        return carry

    lax.fori_loop(0, CPT // IB, block, 0)
    plsc.subcore_barrier()
    pltpu.sync_copy(acc.at[pl.ds(s * RPS, RPS)],
                    out_h.at[c, pl.ds(s * RPS, RPS)])


_agg_edge = pl.kernel(
    _agg_edge_body,
    out_type=jax.ShapeDtypeStruct((NC, NPAD, DH), jnp.float32),
    mesh=_mesh(),
    scratch_types=[
        pltpu.VMEM((IB, CH), jnp.int32),
        pltpu.VMEM((IB, CH), jnp.int32),
        pltpu.VMEM((2, CH, DH), jnp.float32),
        pltpu.SemaphoreType.DMA,
        pltpu.SemaphoreType.DMA,
        pltpu.VMEM_SHARED((NPAD, DH), jnp.float32),
    ],
)


def _tc1_body(deg_ref, x_ref, w_ref, h_ref, gx_ref):
    deg = deg_ref[...]
    dis = lax.rsqrt(deg[:, 0] + deg[:, 1] + 1.0)
    x = x_ref[...]
    h_ref[...] = jnp.dot(x, w_ref[...], preferred_element_type=jnp.float32,
                         precision=lax.Precision.HIGHEST)
    gx_ref[...] = x * dis[:, None]


_tc1_call = pl.pallas_call(
    _tc1_body,
    grid=(N // BM,),
    in_specs=[
        pl.BlockSpec((BM, 2), lambda i: (i, 0)),
        pl.BlockSpec((BM, D_IN), lambda i: (i, 0)),
        pl.BlockSpec((D_IN, D_HID), lambda i: (0, 0)),
    ],
    out_specs=[
        pl.BlockSpec((BM, D_HID), lambda i: (i, 0)),
        pl.BlockSpec((BM, D_IN), lambda i: (i, 0)),
    ],
    out_shape=[
        jax.ShapeDtypeStruct((N, D_HID), jnp.float32),
        jax.ShapeDtypeStruct((N, D_IN), jnp.float32),
    ],
)


def _tc2_body(deg_ref, acc_ref, h1_ref, b1_ref, w1_ref, w2_ref,
              h2_ref, g2_ref):
    deg = deg_ref[...]
    dis = lax.rsqrt(deg[:, 0] + deg[:, 1] + 1.0)
    accx = acc_ref[0, :, :] + acc_ref[1, :, :]
    hagg = jnp.dot(accx, w1_ref[...], preferred_element_type=jnp.float32,
                   precision=lax.Precision.HIGHEST)
    z = hagg * dis[:, None] + h1_ref[...] * (dis * dis)[:, None] + b1_ref[...]
    z = jnp.maximum(z, 0.0)
    h2 = jnp.dot(z, w2_ref[...], preferred_element_type=jnp.float32,
                 precision=lax.Precision.HIGHEST)
    h2_ref[...] = h2
    g2_ref[...] = h2 * dis[:, None]


_tc2_call = pl.pallas_call(
    _tc2_body,
    grid=(N // BM,),
    in_specs=[
        pl.BlockSpec((BM, 2), lambda i: (i, 0)),
        pl.BlockSpec((NC, BM, D_IN), lambda i: (0, i, 0)),
        pl.BlockSpec((BM, D_HID), lambda i: (i, 0)),
        pl.BlockSpec((1, D_HID), lambda i: (0, 0)),
        pl.BlockSpec((D_IN, D_HID), lambda i: (0, 0)),
        pl.BlockSpec((D_HID, D_OUT), lambda i: (0, 0)),
    ],
    out_specs=[
        pl.BlockSpec((BM, D_OUT), lambda i: (i, 0)),
        pl.BlockSpec((BM, D_OUT), lambda i: (i, 0)),
    ],
    out_shape=[
        jax.ShapeDtypeStruct((N, D_OUT), jnp.float32),
        jax.ShapeDtypeStruct((N, D_OUT), jnp.float32),
    ],
)


def _tc3_body(deg_ref, acc_ref, h2_ref, b2_ref, o_ref):
    deg = deg_ref[...]
    dis = lax.rsqrt(deg[:, 0] + deg[:, 1] + 1.0)
    accf = acc_ref[0, :, :] + acc_ref[1, :, :]
    o_ref[...] = (accf * dis[:, None]
                  + h2_ref[...] * (dis * dis)[:, None] + b2_ref[...])


_tc3_call = pl.pallas_call(
    _tc3_body,
    grid=(N // BM,),
    in_specs=[
        pl.BlockSpec((BM, 2), lambda i: (i, 0)),
        pl.BlockSpec((NC, BM, D_OUT), lambda i: (0, i, 0)),
        pl.BlockSpec((BM, D_OUT), lambda i: (i, 0)),
        pl.BlockSpec((1, D_OUT), lambda i: (0, 0)),
    ],
    out_specs=pl.BlockSpec((BM, D_OUT), lambda i: (i, 0)),
    out_shape=jax.ShapeDtypeStruct((N, D_OUT), jnp.float32),
)


@jax.jit
def kernel(x, edge_index, W1, b1, W2, b2):
    src = edge_index[0].astype(jnp.int32)
    dst = edge_index[1].astype(jnp.int32)
    pad = E_PAD - E
    dst_p = jnp.concatenate(
        [dst, jnp.full((pad,), N, jnp.int32)]).reshape(NCHUNK, CH)
    src_p = jnp.concatenate(
        [src, jnp.zeros((pad,), jnp.int32)]).reshape(NCHUNK, CH)

    ones_h = jnp.ones((CH,), jnp.float32)
    zeros_deg = jnp.zeros((NPAD,), jnp.float32)
    zeros128 = jnp.zeros((NPAD, DH), jnp.float32)

    degp = _deg_call(dst_p, ones_h, zeros_deg)
    deg_nt = jnp.transpose(degp[:, :N], (1, 0))

    h1, gx = _tc1_call(deg_nt, x, W1)
    acc1 = _agg_edge(gx, src_p, dst_p, zeros128)
    h2, g2 = _tc2_call(deg_nt, acc1, h1, b1.reshape(1, D_HID), W1, W2)
    acc2 = _agg_edge(g2, src_p, dst_p, zeros128)
    out = _tc3_call(deg_nt, acc2, h2, b2.reshape(1, D_OUT))
    return out
